# Initial kernel scaffold; baseline (speedup 1.0000x reference)
import jax
import jax.numpy as jnp
from jax.experimental import pallas as pl


def kernel(node_emb, feature_emb, relation_index, W1, b1, g1, be1, W2, b2, g2, be2, W3, b3):
    raise NotImplementedError("write your pallas kernel here")


# speedup vs baseline: 13.4631x; 13.4631x over previous
import jax
import jax.numpy as jnp
from jax.experimental import pallas as pl


def _head_body(node_ref, feat_ref, rel_ref, W1_ref, b1_ref, g1_ref, be1_ref,
               W2_ref, b2_ref, g2_ref, be2_ref, W3_ref, b3_ref, out_ref):
    nf = feat_ref.shape[0]
    ne = rel_ref.shape[1]
    d = feat_ref.shape[1]

    f_iota = jax.lax.broadcasted_iota(jnp.int32, (nf, ne), 0)
    S = (rel_ref[0:1, :] == f_iota).astype(jnp.float32)
    Dh = (rel_ref[1:2, :] == f_iota).astype(jnp.float32)
    C = jnp.dot(S, Dh.T, preferred_element_type=jnp.float32)
    counts = 1.0 + jnp.sum(S, axis=1, keepdims=True)
    feat = feat_ref[...]
    msg = (feat + jnp.dot(C, feat, preferred_element_type=jnp.float32)) / counts

    node = node_ref[...]
    A = jnp.dot(node, W1_ref[:, :d].T, preferred_element_type=jnp.float32)
    B = jnp.dot(msg, W1_ref[:, d:].T, preferred_element_type=jnp.float32) + b1_ref[...]

    mA = jnp.mean(A, axis=0, keepdims=True)
    vA = jnp.mean((A - mA) ** 2, axis=0, keepdims=True)
    mB = jnp.mean(B, axis=0, keepdims=True)
    vB = jnp.mean((B - mB) ** 2, axis=0, keepdims=True)
    s1 = g1_ref[...] * jax.lax.rsqrt(vA + vB + 1e-5)
    A1 = A * s1
    B1 = (B - mA - mB) * s1 + be1_ref[...]

    A2 = jnp.dot(A1, W2_ref[...].T, preferred_element_type=jnp.float32)
    B2 = jnp.dot(B1, W2_ref[...].T, preferred_element_type=jnp.float32) + b2_ref[...]
    mA2 = jnp.mean(A2, axis=0, keepdims=True)
    vA2 = jnp.mean((A2 - mA2) ** 2, axis=0, keepdims=True)
    mB2 = jnp.mean(B2, axis=0, keepdims=True)
    vB2 = jnp.mean((B2 - mB2) ** 2, axis=0, keepdims=True)
    s2 = g2_ref[...] * jax.lax.rsqrt(vA2 + vB2 + 1e-5)
    A2p = A2 * s2
    B2p = (B2 - mA2 - mB2) * s2 + be2_ref[...]

    a3 = jnp.sum(A2p * W3_ref[...], axis=1, keepdims=True)
    bf3 = jnp.sum(B2p * W3_ref[...], axis=1, keepdims=True) + b3_ref[...]
    out_ref[...] = a3 + bf3.T


def kernel(node_emb, feature_emb, relation_index, W1, b1, g1, be1, W2, b2, g2, be2, W3, b3):
    bs, d = node_emb.shape
    nf = feature_emb.shape[0]
    out = pl.pallas_call(
        _head_body,
        out_shape=jax.ShapeDtypeStruct((bs, nf), jnp.float32),
    )(node_emb, feature_emb, relation_index,
      W1, b1.reshape(1, -1), g1.reshape(1, -1), be1.reshape(1, -1),
      W2, b2.reshape(1, -1), g2.reshape(1, -1), be2.reshape(1, -1),
      W3, b3.reshape(1, 1))
    return out
